# Initial kernel scaffold; baseline (speedup 1.0000x reference)
import jax
import jax.numpy as jnp
from jax.experimental import pallas as pl


def kernel(h, edge_index):
    raise NotImplementedError("write your pallas kernel here")


# speedup vs baseline: 1.1401x; 1.1401x over previous
import functools

import jax
import jax.numpy as jnp
from jax import lax
from jax.experimental import pallas as pl
from jax.experimental.pallas import tpu as pltpu
from jax.experimental.pallas import tpu_sc as plsc

N_NODES_ = 10000
N_EDGES_ = 320000
D_ = 128

NC = 2
NS = 16
NW = NC * NS

CH = 128
CPW = 80
EPW = CPW * CH
EPAD = EPW * NW
NBUF = 2


def _make_sc_kernel():
    mesh = plsc.VectorSubcoreMesh(core_axis_name="c", subcore_axis_name="s")

    @functools.partial(
        pl.kernel,
        mesh=mesh,
        out_type=jax.ShapeDtypeStruct((EPAD,), jnp.float32),
        compiler_params=pltpu.CompilerParams(needs_layout_passes=False),
        scratch_types=[
            pltpu.VMEM((EPW,), jnp.int32),
            pltpu.VMEM((EPW,), jnp.int32),
            pltpu.VMEM((CH, D_), jnp.float32),
            pltpu.VMEM((CH, D_), jnp.float32),
            pltpu.VMEM((CH, D_), jnp.float32),
            pltpu.VMEM((CH, D_), jnp.float32),
            pltpu.VMEM((EPW,), jnp.float32),
            pltpu.SemaphoreType.DMA,
            pltpu.SemaphoreType.DMA,
        ],
    )
    def sc_kernel(ei_hbm, h_hbm, out_hbm, idx_s, idx_d, rows_s0, rows_s1,
                  rows_d0, rows_d1, scores, sem0, sem1):
        rows_s = (rows_s0, rows_s1)
        rows_d = (rows_d0, rows_d1)
        wid = lax.axis_index("s") * NC + lax.axis_index("c")
        base = wid * EPW

        pltpu.sync_copy(ei_hbm.at[0, pl.ds(base, EPW)], idx_s)
        pltpu.sync_copy(ei_hbm.at[1, pl.ds(base, EPW)], idx_d)

        sems = (sem0, sem1)

        def issue(c, slot):
            pltpu.async_copy(
                h_hbm.at[idx_s.at[pl.ds(c * CH, CH)]], rows_s[slot],
                sems[slot])
            pltpu.async_copy(
                h_hbm.at[idx_d.at[pl.ds(c * CH, CH)]], rows_d[slot],
                sems[slot])

        def wait(slot):
            pltpu.make_async_copy(
                h_hbm.at[idx_s.at[pl.ds(0, CH)]], rows_s[slot],
                sems[slot]).wait()
            pltpu.make_async_copy(
                h_hbm.at[idx_d.at[pl.ds(0, CH)]], rows_d[slot],
                sems[slot]).wait()

        def compute(c, slot):
            def grp(g, carry):
                e0 = g * 16
                idx_e = e0 + lax.iota(jnp.int32, 16)
                accs = [jnp.zeros((16,), jnp.float32) for _ in range(4)]
                for d in range(D_):
                    col = jnp.full((16,), d, jnp.int32)
                    a = plsc.load_gather(rows_s[slot], [idx_e, col])
                    b = plsc.load_gather(rows_d[slot], [idx_e, col])
                    accs[d % 4] = accs[d % 4] + a * b
                acc = (accs[0] + accs[1]) + (accs[2] + accs[3])
                scores[pl.ds(c * CH + e0, 16)] = acc
                return carry
            lax.fori_loop(0, CH // 16, grp, 0)

        issue(0, 0)

        def chunk_pair(i, carry):
            c0 = 2 * i
            issue(c0 + 1, 1)
            wait(0)
            compute(c0, 0)
            issue(jnp.minimum(c0 + 2, CPW - 1), 0)
            wait(1)
            compute(c0 + 1, 1)
            return carry

        lax.fori_loop(0, CPW // 2, chunk_pair, 0)
        wait(0)

        pltpu.sync_copy(scores, out_hbm.at[pl.ds(base, EPW)])

    return sc_kernel


_sc_kernel = _make_sc_kernel()


@jax.jit
def kernel(h, edge_index):
    ei = edge_index.astype(jnp.int32)
    ei = jnp.pad(ei, ((0, 0), (0, EPAD - N_EDGES_)))
    out = _sc_kernel(ei, h)
    return out[:N_EDGES_, None]
